# Initial kernel scaffold; baseline (speedup 1.0000x reference)
import jax
import jax.numpy as jnp
from jax.experimental import pallas as pl


def kernel(x, edge_index, edge_weight, W):
    raise NotImplementedError("write your pallas kernel here")


# speedup vs baseline: 3.6735x; 3.6735x over previous
import functools

import jax
import jax.numpy as jnp
from jax import lax
from jax.experimental import pallas as pl
from jax.experimental.pallas import tpu as pltpu
from jax.experimental.pallas import tpu_sc as plsc

NC = 2
NS = 16
L = 16

CHUNK = 128
NBUF = 2


def _sc_scatter(n_nodes, d, nchunk):
    rows_per_tile = n_nodes // NS
    ngroups = d // L
    blk = -(-rows_per_tile // 8) * 8

    mesh = plsc.VectorSubcoreMesh(core_axis_name="c", subcore_axis_name="s")

    @functools.partial(
        pl.kernel,
        out_type=jax.ShapeDtypeStruct((NC, n_nodes, d), jnp.float32),
        mesh=mesh,
        scratch_types=dict(
            src_v=pltpu.VMEM((NBUF, CHUNK), jnp.int32),
            dst_v=pltpu.VMEM((NBUF, CHUNK), jnp.int32),
            w_v=pltpu.VMEM((NBUF, CHUNK), jnp.float32),
            rows=pltpu.VMEM((NBUF, CHUNK, d), jnp.float32),
            acc=pltpu.VMEM_SHARED((n_nodes, d), jnp.float32),
            semg0=pltpu.SemaphoreType.DMA,
            semg1=pltpu.SemaphoreType.DMA,
            semi0=pltpu.SemaphoreType.DMA,
            semi1=pltpu.SemaphoreType.DMA,
        ),
    )
    def sc_kernel(x_hbm, src_hbm, dst_hbm, w_hbm, out_hbm,
                  src_v, dst_v, w_v, rows, acc, semg0, semg1, semi0, semi1):
        c = lax.axis_index("c")
        s = lax.axis_index("s")
        wid = c * NS + s
        semg = (semg0, semg1)
        semi = (semi0, semi1)

        def fire_idx(j, b):
            pltpu.async_copy(src_hbm.at[wid, j], src_v.at[b], semi[b])
            pltpu.async_copy(dst_hbm.at[wid, j], dst_v.at[b], semi[b])
            pltpu.async_copy(w_hbm.at[wid, j], w_v.at[b], semi[b])

        def wait_idx(j, b):
            pltpu.make_async_copy(src_hbm.at[wid, j], src_v.at[b], semi[b]).wait()
            pltpu.make_async_copy(dst_hbm.at[wid, j], dst_v.at[b], semi[b]).wait()
            pltpu.make_async_copy(w_hbm.at[wid, j], w_v.at[b], semi[b]).wait()

        r0 = rows.at[0]

        def _zero_row(i, _):
            for g in range(ngroups):
                r0[i, pl.ds(g * L, L)] = jnp.zeros((L,), jnp.float32)
            return 0

        lax.fori_loop(0, CHUNK, _zero_row, 0)
        a = jnp.minimum((s * rows_per_tile) // 8 * 8, n_nodes - blk)
        off = 0
        for h in [CHUNK] * (blk // CHUNK) + ([blk % CHUNK] if blk % CHUNK else []):
            pltpu.sync_copy(r0.at[pl.ds(0, h)], acc.at[pl.ds(a + off, h)])
            off += h
        plsc.subcore_barrier()

        def _chunk(j, _):
            b = 0
            rb = rows.at[b]
            pltpu.sync_copy(src_hbm.at[wid, j], src_v.at[b])
            pltpu.sync_copy(dst_hbm.at[wid, j], dst_v.at[b])
            pltpu.sync_copy(w_hbm.at[wid, j], w_v.at[b])
            pltpu.async_copy(x_hbm.at[src_v.at[b]], rb, semg[b]).wait()

            def _scale16(t, _):
                wv = w_v[b, pl.ds(t * L, L)]
                for ee in range(L):
                    wb = jnp.broadcast_to(wv[ee], (L,))
                    e = t * L + ee
                    for g in range(ngroups):
                        rb[e, pl.ds(g * L, L)] = rb[e, pl.ds(g * L, L)] * wb
                return 0

            lax.fori_loop(0, CHUNK // L, _scale16, 0)

            pltpu.sync_copy(rb, acc.at[dst_v.at[b]], add=True)
            return 0

        lax.fori_loop(0, nchunk, _chunk, 0)

        plsc.subcore_barrier()
        pltpu.sync_copy(acc.at[pl.ds(a, blk)],
                        out_hbm.at[c].at[pl.ds(a, blk)])

    return sc_kernel


def _mm_body(a_ref, w_ref, o_ref):
    s = a_ref[0] + a_ref[1]
    o_ref[...] = jnp.dot(s, w_ref[...], preferred_element_type=jnp.float32)


def kernel(x, edge_index, edge_weight, W):
    n, d = x.shape
    e = edge_weight.shape[0]
    nw = NC * NS

    per_w = -(-e // (nw * CHUNK)) * CHUNK
    e_pad = per_w * nw
    pad = e_pad - e
    src = jnp.pad(edge_index[0], (0, pad)).reshape(nw, per_w // CHUNK, CHUNK)
    dst = jnp.pad(edge_index[1], (0, pad)).reshape(nw, per_w // CHUNK, CHUNK)
    w_e = jnp.pad(edge_weight, (0, pad)).reshape(nw, per_w // CHUNK, CHUNK)

    partial = _sc_scatter(n, d, per_w // CHUNK)(x, src, dst, w_e)

    rows_blk = 1000
    out = pl.pallas_call(
        _mm_body,
        grid=(n // rows_blk,),
        in_specs=[
            pl.BlockSpec((NC, rows_blk, d), lambda i: (0, i, 0)),
            pl.BlockSpec((d, d), lambda i: (0, 0)),
        ],
        out_specs=pl.BlockSpec((rows_blk, d), lambda i: (i, 0)),
        out_shape=jax.ShapeDtypeStruct((n, d), jnp.float32),
    )(partial, W)
    return out
